# baseline (speedup 1.0000x reference)
import functools

import jax
import jax.numpy as jnp
from jax import lax
from jax.experimental import pallas as pl
from jax.experimental.pallas import tpu as pltpu
from jax.experimental.pallas import tpu_sc as plsc

N_TOKENS = 16384
CODEBOOK_SIZE = 8192
CODE_DIM = 32

TN = 1024
KC = 2048


def _argmax_body(x_ref, cb_ref, idx_ref, max_sc, idx_sc):
    j = pl.program_id(1)
    x = x_ref[...]
    cb = cb_ref[...]
    prod = lax.dot_general(
        x, cb, (((1,), (1,)), ((), ())),
        preferred_element_type=jnp.float32)
    x_sq = jnp.sum(x * x, axis=1, keepdims=True)
    e_sq = jnp.sum(cb * cb, axis=1)[None, :]
    dist = -2.0 * prod + x_sq + e_sq
    cmax = jnp.max(dist, axis=1)
    ids = lax.broadcasted_iota(jnp.int32, dist.shape, 1)
    hit = jnp.where(dist == cmax[:, None], ids, CODEBOOK_SIZE)
    carg = jnp.min(hit, axis=1) + j * KC

    @pl.when(j == 0)
    def _():
        max_sc[...] = cmax
        idx_sc[...] = carg

    @pl.when(j > 0)
    def _():
        better = cmax > max_sc[...]
        idx_sc[...] = jnp.where(better, carg, idx_sc[...])
        max_sc[...] = jnp.maximum(cmax, max_sc[...])

    @pl.when(j == pl.num_programs(1) - 1)
    def _():
        idx_ref[...] = idx_sc[...]


def _argmax_call(x, codebook):
    grid = (N_TOKENS // TN, CODEBOOK_SIZE // KC)
    return pl.pallas_call(
        _argmax_body,
        grid=grid,
        in_specs=[
            pl.BlockSpec((TN, CODE_DIM), lambda i, j: (i, 0)),
            pl.BlockSpec((KC, CODE_DIM), lambda i, j: (j, 0)),
        ],
        out_specs=pl.BlockSpec((TN,), lambda i, j: (i,)),
        out_shape=jax.ShapeDtypeStruct((N_TOKENS,), jnp.int32),
        scratch_shapes=[
            pltpu.VMEM((TN,), jnp.float32),
            pltpu.VMEM((TN,), jnp.int32),
        ],
        compiler_params=pltpu.CompilerParams(
            dimension_semantics=("parallel", "arbitrary")),
    )(x, codebook)


_NC = 2
_NS = 16
_NW = _NC * _NS
_BPW = N_TOKENS // _NW
_CHUNK = 128
_NCHUNK = _BPW // _CHUNK


def _gather_body(cb_hbm, idx_hbm, out_hbm, idx_v, rows_v, sem):
    wid = lax.axis_index("s") * _NC + lax.axis_index("c")
    base = wid * _BPW
    pltpu.sync_copy(idx_hbm.at[pl.ds(base, _BPW)], idx_v)
    copies = [
        pltpu.async_copy(
            cb_hbm.at[idx_v.at[pl.ds(c * _CHUNK, _CHUNK)]],
            rows_v.at[pl.ds(c * _CHUNK, _CHUNK)],
            sem,
        )
        for c in range(_NCHUNK)
    ]
    for c in copies:
        c.wait()
    pltpu.sync_copy(rows_v, out_hbm.at[pl.ds(base, _BPW)])


def _gather_call(codebook, idx):
    mesh = plsc.VectorSubcoreMesh(core_axis_name="c", subcore_axis_name="s")
    f = pl.kernel(
        _gather_body,
        mesh=mesh,
        out_type=jax.ShapeDtypeStruct((N_TOKENS, CODE_DIM), jnp.float32),
        scratch_types=[
            pltpu.VMEM((_BPW,), jnp.int32),
            pltpu.VMEM((_BPW, CODE_DIM), jnp.float32),
            pltpu.SemaphoreType.DMA,
        ],
        compiler_params=pltpu.CompilerParams(use_tc_tiling_on_sc=False),
    )
    return f(codebook, idx)


def kernel(x, codebook):
    best_idx = _argmax_call(x, codebook)
    return _gather_call(codebook, best_idx)
